# baseline (speedup 1.0000x reference)
import functools

import jax
import jax.numpy as jnp
from jax import lax
from jax.experimental import pallas as pl
from jax.experimental.pallas import tpu as pltpu
from jax.experimental.pallas import tpu_sc as plsc

_EMBED_DIM = 16
_INPUT_DIM = 64
_HIDDEN = 64
_BATCH = 16384
_VROWS = 12500

_NC = 2
_NS = 16
_NW = _NC * _NS
_BPW = _BATCH // _NW
_CHUNK = 128
_NCHUNK = _BPW // _CHUNK
_L = 16
_PACK = _BATCH // 2


def _sc_gather(table128, idx3):
    mesh = plsc.VectorSubcoreMesh(core_axis_name="c", subcore_axis_name="s")

    @functools.partial(
        pl.kernel,
        mesh=mesh,
        compiler_params=pltpu.CompilerParams(use_tc_tiling_on_sc=False,
                                             needs_layout_passes=False),
        out_type=jax.ShapeDtypeStruct((_PACK, 128), jnp.float32),
        scratch_types=[
            pltpu.VMEM((_NCHUNK, _CHUNK), jnp.int32),
            pltpu.VMEM((_NCHUNK, _CHUNK), jnp.int32),
            pltpu.VMEM((_BPW, 128), jnp.float32),
            pltpu.VMEM((_BPW // 2, 128), jnp.float32),
            pltpu.SemaphoreType.DMA,
        ],
    )
    def gather_kernel(table_hbm, idx_hbm, out_hbm, idx_v, vidx_v, rows_v,
                      emb_v, sem):
        wid = lax.axis_index("s") * _NC + lax.axis_index("c")
        pltpu.sync_copy(idx_hbm.at[wid], idx_v)
        for j in range(_NCHUNK):
            for t in range(_CHUNK // _L):
                sl = pl.ds(t * _L, _L)
                vidx_v[j, sl] = lax.shift_right_logical(idx_v[j, sl], 3)
        copies = [
            pltpu.async_copy(
                table_hbm.at[vidx_v.at[j]],
                rows_v.at[pl.ds(j * _CHUNK, _CHUNK)],
                sem,
            )
            for j in range(_NCHUNK)
        ]
        for cp in copies:
            cp.wait()
        lane = lax.iota(jnp.int32, _L)
        pos_base = (lane & 1) * 64
        half = lax.shift_right_logical(lane, 1)
        for g in range(_BPW // _L):
            sub = idx_v[g // 8, pl.ds((g % 8) * _L, _L)] & 7
            o_vec = sub * _EMBED_DIM
            r_vec = lane + g * _L
            c_vec = half + g * (_L // 2)
            for j in range(_EMBED_DIM):
                vals = plsc.load_gather(rows_v, [r_vec, o_vec + j])
                plsc.store_scatter(emb_v, [c_vec, pos_base + j], vals)
        pltpu.sync_copy(emb_v, out_hbm.at[pl.ds(wid * (_BPW // 2), _BPW // 2)])

    return gather_kernel(table128, idx3)


_BR2 = 1024


def _mlp_body(emb_ref, feat_ref, w1_ref, b1_ref, w2t_ref, out_ref):
    w1e = w1_ref[0:_EMBED_DIM, :]
    w1f = w1_ref[_EMBED_DIM:, :]
    b1 = b1_ref[...]
    w2t = w2t_ref[...]
    feat = feat_ref[...]
    emb = emb_ref[...]
    outs = []
    for p in range(2):
        f = feat[:, 64 * p:64 * p + 64]
        e = emb[:, 64 * p:64 * p + _EMBED_DIM]
        x = jnp.dot(f, w1f, preferred_element_type=jnp.float32)
        x = x + jnp.dot(e, w1e, preferred_element_type=jnp.float32)
        h = jnp.maximum(x + b1, 0.0)
        o = jnp.sum(h * w2t, axis=1, keepdims=True)
        outs.append(o)
        outs.append(jnp.zeros((_BR2, 63), jnp.float32))
    out_ref[...] = jnp.concatenate(outs, axis=1)


def _tc_mlp(emb128, feat128, W1, b1r, w2t, interpret=False):
    grid = (_PACK // _BR2,)
    return pl.pallas_call(
        _mlp_body,
        grid=grid,
        in_specs=[
            pl.BlockSpec((_BR2, 128), lambda i: (i, 0)),
            pl.BlockSpec((_BR2, 128), lambda i: (i, 0)),
            pl.BlockSpec((_EMBED_DIM + _INPUT_DIM, _HIDDEN), lambda i: (0, 0)),
            pl.BlockSpec((1, _HIDDEN), lambda i: (0, 0)),
            pl.BlockSpec((1, _HIDDEN), lambda i: (0, 0)),
        ],
        out_specs=pl.BlockSpec((_BR2, 128), lambda i: (i, 0)),
        out_shape=jax.ShapeDtypeStruct((_PACK, 128), jnp.float32),
        interpret=interpret,
    )(emb128, feat128, W1, b1r, w2t)


def kernel(pitcher_id, features, table, W1, b1, W2, b2):
    pid = pitcher_id.astype(jnp.int32)
    idx3 = pid.reshape(_NW, _NCHUNK, _CHUNK)
    table128 = table.reshape(_VROWS, 128)
    feat128 = features.reshape(_PACK, 128)
    emb128 = _sc_gather(table128, idx3)
    b1r = b1.reshape(1, _HIDDEN)
    w2t = W2.reshape(1, _HIDDEN)
    out128 = _tc_mlp(emb128, feat128, W1, b1r, w2t)
    out = out128.reshape(_PACK, 2, 64)[:, :, 0].reshape(_BATCH, 1)
    return out + b2
